# baseline (speedup 1.0000x reference)
import functools

import jax
import jax.numpy as jnp
from jax import lax
from jax.experimental import pallas as pl
from jax.experimental.pallas import tpu as pltpu
from jax.experimental.pallas import tpu_sc as plsc

_VOCAB = 1000000
_EMBED = 32
_NUM_CLASS = 16
_MAX_LEN = 200
_BATCH = 4096

_TOTAL = _BATCH * _MAX_LEN
_NC, _NS = 2, 16
_NW = _NC * _NS
_PER_W = _TOTAL // _NW
_CHUNK = 1280
_NCHUNK = _PER_W // _CHUNK


def _gather_body(idx_hbm, table_hbm, out_hbm, idx_all, rows_v, gsem, osem):
    wid = lax.axis_index("s") * _NC + lax.axis_index("c")
    base = wid * _PER_W
    pltpu.sync_copy(idx_hbm.at[pl.ds(base, _PER_W)], idx_all)

    def g_copy(i):
        b = i % 2
        return pltpu.make_async_copy(
            table_hbm.at[idx_all.at[pl.ds(i * _CHUNK, _CHUNK)]],
            rows_v.at[b], gsem.at[b])

    def o_copy(i):
        b = i % 2
        return pltpu.make_async_copy(
            rows_v.at[b], out_hbm.at[pl.ds(base + i * _CHUNK, _CHUNK)],
            osem.at[b])

    g_copy(0).start()
    for i in range(_NCHUNK):
        g_copy(i).wait()
        if i + 1 < _NCHUNK:
            if i >= 1:
                o_copy(i - 1).wait()
            g_copy(i + 1).start()
        o_copy(i).start()
    o_copy(_NCHUNK - 2).wait()
    o_copy(_NCHUNK - 1).wait()


_sc_gather = functools.partial(
    pl.kernel,
    out_type=jax.ShapeDtypeStruct((_TOTAL, _EMBED), jnp.float32),
    mesh=plsc.VectorSubcoreMesh(core_axis_name="c", subcore_axis_name="s"),
    scratch_types=[
        pltpu.VMEM((_PER_W,), jnp.int32),
        pltpu.VMEM((2, _CHUNK, _EMBED), jnp.float32),
        pltpu.SemaphoreType.DMA((2,)),
        pltpu.SemaphoreType.DMA((2,)),
    ],
    compiler_params=pltpu.CompilerParams(use_tc_tiling_on_sc=False),
)(_gather_body)


_TCOLS = 7812
_VTAIL = _VOCAB - _TCOLS * 128


def _tr_body(t32_hbm, tail_hbm, out_hbm, in_v, out_v, isem, osem):
    wid = lax.axis_index("s") * _NC + lax.axis_index("c")
    nu = jnp.where(wid < _TCOLS % _NW, _TCOLS // _NW + 1, _TCOLS // _NW)
    iota_a = lax.iota(jnp.int32, 16)
    iota_b = iota_a + 16

    def in_copy(i, b):
        unit = wid + i * _NW
        return pltpu.make_async_copy(
            t32_hbm.at[:, pl.ds(unit * 128, 128)], in_v.at[b], isem.at[b])

    def out_copy(i, b):
        unit = wid + i * _NW
        return pltpu.make_async_copy(
            out_v.at[b], out_hbm.at[pl.ds(unit * 32, 32), :], osem.at[b])

    def transpose_block(b, width):
        def row_blk(c2, _):
            for k in range(8):
                cvec = jnp.full((16,), c2 * 8 + k, dtype=jnp.int32)
                ga = plsc.load_gather(in_v.at[b], [iota_a, cvec])
                gb = plsc.load_gather(in_v.at[b], [iota_b, cvec])
                row = c2 * 2 + k // 4
                cb = (k % 4) * 32
                out_v[b, row, pl.ds(cb, 16)] = ga
                out_v[b, row, pl.ds(cb + 16, 16)] = gb
            return 0
        lax.fori_loop(0, width // 8, row_blk, 0)

    in_copy(0, 0).start()

    def body(i, _):
        b = i % 2
        in_copy(i, b).wait()

        @pl.when(i + 1 < nu)
        def _():
            in_copy(i + 1, 1 - b).start()

        @pl.when(i >= 2)
        def _():
            out_copy(i - 2, b).wait()

        transpose_block(b, 128)
        out_copy(i, b).start()
        return 0

    lax.fori_loop(0, nu, body, 0)

    @pl.when(nu >= 2)
    def _():
        out_copy(nu - 2, nu % 2).wait()
    out_copy(nu - 1, (nu - 1) % 2).wait()

    @pl.when(wid == 0)
    def _():
        pltpu.sync_copy(tail_hbm, in_v.at[0, pl.ds(0, 16), :])
        pltpu.sync_copy(in_v.at[0, pl.ds(0, 16), :],
                        out_hbm.at[pl.ds(_TCOLS * 32, 16), :])


_sc_transpose = functools.partial(
    pl.kernel,
    out_type=jax.ShapeDtypeStruct((_VOCAB // 4, 128), jnp.float32),
    mesh=plsc.VectorSubcoreMesh(core_axis_name="c", subcore_axis_name="s"),
    scratch_types=[
        pltpu.VMEM((2, _EMBED, 128), jnp.float32),
        pltpu.VMEM((2, 32, 128), jnp.float32),
        pltpu.SemaphoreType.DMA((2,)),
        pltpu.SemaphoreType.DMA((2,)),
    ],
    compiler_params=pltpu.CompilerParams(
        use_tc_tiling_on_sc=True, needs_layout_passes=False),
)(_tr_body)


_LB = 8
_I4 = _BATCH // 4


def _mm_body(x_ref, w_ref, b_ref, o_ref):
    @pl.when(pl.program_id(0) == 0)
    def _():
        o_ref[...] = jnp.broadcast_to(b_ref[...], (_I4, 4 * _NUM_CLASS))

    acc = lax.dot_general(
        x_ref[0], w_ref[0], (((1,), (0,)), ((), ())),
        preferred_element_type=jnp.float32,
    )
    for j in range(1, _LB):
        acc += lax.dot_general(
            x_ref[j], w_ref[j], (((1,), (0,)), ((), ())),
            preferred_element_type=jnp.float32,
        )
    o_ref[...] += acc


def _tc_matmul(x3, w4, b4):
    return pl.pallas_call(
        _mm_body,
        grid=(_MAX_LEN // _LB,),
        in_specs=[
            pl.BlockSpec((_LB, _I4, 128), lambda i: (i, 0, 0)),
            pl.BlockSpec((_LB, 128, 4 * _NUM_CLASS), lambda i: (i, 0, 0)),
            pl.BlockSpec((1, 4 * _NUM_CLASS), lambda i: (0, 0)),
        ],
        out_specs=pl.BlockSpec((_I4, 4 * _NUM_CLASS), lambda i: (0, 0)),
        out_shape=jax.ShapeDtypeStruct((_I4, 4 * _NUM_CLASS), jnp.float32),
    )(x3, w4, b4)


@jax.jit
def kernel(text, emb_table, fc_w, fc_b):
    flat_idx = text.T.reshape(_TOTAL).astype(jnp.int32)
    tail = emb_table[_TCOLS * 128:, :].reshape(16, 128)
    tab_lin = _sc_transpose(emb_table.T, tail).reshape(_VOCAB, _EMBED)
    rows = _sc_gather(flat_idx, tab_lin)
    x3 = rows.reshape(_MAX_LEN, _I4, 128)
    wl = fc_w.reshape(_NUM_CLASS, _MAX_LEN, _EMBED).transpose(1, 2, 0)
    eye4 = jnp.eye(4, dtype=jnp.float32)
    w4 = (wl[:, None, :, None, :] * eye4[None, :, None, :, None]).reshape(
        _MAX_LEN, 128, 4 * _NUM_CLASS)
    b4 = jnp.tile(fc_b, 4).reshape(1, 4 * _NUM_CLASS)
    out4 = _tc_matmul(x3, w4, b4)
    return out4.reshape(_BATCH, _NUM_CLASS)


# speedup vs baseline: 1.3650x; 1.3650x over previous
import functools

import jax
import jax.numpy as jnp
from jax import lax
from jax.experimental import pallas as pl
from jax.experimental.pallas import tpu as pltpu
from jax.experimental.pallas import tpu_sc as plsc

_VOCAB = 1000000
_EMBED = 32
_NUM_CLASS = 16
_MAX_LEN = 200
_BATCH = 4096

_TOTAL = _BATCH * _MAX_LEN
_NC, _NS = 2, 16
_NW = _NC * _NS
_PER_W = _TOTAL // _NW
_CHUNK = 1280
_NCHUNK = _PER_W // _CHUNK


def _gather_body(idx_hbm, table_hbm, out_hbm, idx_all, rows_v, gsem, osem):
    wid = lax.axis_index("s") * _NC + lax.axis_index("c")
    base = wid * _PER_W
    pltpu.sync_copy(idx_hbm.at[pl.ds(base, _PER_W)], idx_all)

    def g_copy(i):
        b = i % 2
        return pltpu.make_async_copy(
            table_hbm.at[idx_all.at[pl.ds(i * _CHUNK, _CHUNK)]],
            rows_v.at[b], gsem.at[b])

    def o_copy(i):
        b = i % 2
        return pltpu.make_async_copy(
            rows_v.at[b], out_hbm.at[pl.ds(base + i * _CHUNK, _CHUNK)],
            osem.at[b])

    g_copy(0).start()
    for i in range(_NCHUNK):
        g_copy(i).wait()
        if i + 1 < _NCHUNK:
            if i >= 1:
                o_copy(i - 1).wait()
            g_copy(i + 1).start()
        o_copy(i).start()
    o_copy(_NCHUNK - 2).wait()
    o_copy(_NCHUNK - 1).wait()


_sc_gather = functools.partial(
    pl.kernel,
    out_type=jax.ShapeDtypeStruct((_TOTAL, _EMBED), jnp.float32),
    mesh=plsc.VectorSubcoreMesh(core_axis_name="c", subcore_axis_name="s"),
    scratch_types=[
        pltpu.VMEM((_PER_W,), jnp.int32),
        pltpu.VMEM((2, _CHUNK, _EMBED), jnp.float32),
        pltpu.SemaphoreType.DMA((2,)),
        pltpu.SemaphoreType.DMA((2,)),
    ],
    compiler_params=pltpu.CompilerParams(use_tc_tiling_on_sc=False),
)(_gather_body)


_TCOLS = 7812
_VTAIL = _VOCAB - _TCOLS * 128


def _tr_body(t32_hbm, tail_hbm, out_hbm, in_v, out_v, isem, osem):
    wid = lax.axis_index("s") * _NC + lax.axis_index("c")
    nu = jnp.where(wid < _TCOLS % _NW, _TCOLS // _NW + 1, _TCOLS // _NW)
    iota_a = lax.iota(jnp.int32, 16)
    iota_b = iota_a + 16

    def in_copy(i, b):
        unit = wid + i * _NW
        return pltpu.make_async_copy(
            t32_hbm.at[:, pl.ds(unit * 128, 128)], in_v.at[b], isem.at[b])

    def out_copy(i, b):
        unit = wid + i * _NW
        return pltpu.make_async_copy(
            out_v.at[b], out_hbm.at[pl.ds(unit * 32, 32), :], osem.at[b])

    def transpose_block(b, width):
        def row_blk(c2, cvec):
            gs = []
            for k in range(16):
                gs.append(plsc.load_gather(in_v.at[b], [iota_a, cvec]))
                gs.append(plsc.load_gather(in_v.at[b], [iota_b, cvec]))
                cvec = cvec + 1
            for k in range(16):
                row = c2 * 4 + k // 4
                cb = (k % 4) * 32
                out_v[b, row, pl.ds(cb, 16)] = gs[2 * k]
                out_v[b, row, pl.ds(cb + 16, 16)] = gs[2 * k + 1]
            return cvec
        lax.fori_loop(0, width // 16, row_blk,
                      jnp.zeros((16,), jnp.int32))

    in_copy(0, 0).start()

    def body(i, _):
        b = i % 2
        in_copy(i, b).wait()

        @pl.when(i + 1 < nu)
        def _():
            in_copy(i + 1, 1 - b).start()

        @pl.when(i >= 2)
        def _():
            out_copy(i - 2, b).wait()

        transpose_block(b, 128)
        out_copy(i, b).start()
        return 0

    lax.fori_loop(0, nu, body, 0)

    @pl.when(nu >= 2)
    def _():
        out_copy(nu - 2, nu % 2).wait()
    out_copy(nu - 1, (nu - 1) % 2).wait()

    @pl.when(wid == 0)
    def _():
        pltpu.sync_copy(tail_hbm, in_v.at[0, pl.ds(0, 16), :])
        pltpu.sync_copy(in_v.at[0, pl.ds(0, 16), :],
                        out_hbm.at[pl.ds(_TCOLS * 32, 16), :])


_sc_transpose = functools.partial(
    pl.kernel,
    out_type=jax.ShapeDtypeStruct((_VOCAB // 4, 128), jnp.float32),
    mesh=plsc.VectorSubcoreMesh(core_axis_name="c", subcore_axis_name="s"),
    scratch_types=[
        pltpu.VMEM((2, _EMBED, 128), jnp.float32),
        pltpu.VMEM((2, 32, 128), jnp.float32),
        pltpu.SemaphoreType.DMA((2,)),
        pltpu.SemaphoreType.DMA((2,)),
    ],
    compiler_params=pltpu.CompilerParams(
        use_tc_tiling_on_sc=True, needs_layout_passes=False),
)(_tr_body)


_LB = 8
_I4 = _BATCH // 4


def _mm_body(x_ref, w_ref, b_ref, o_ref):
    @pl.when(pl.program_id(0) == 0)
    def _():
        o_ref[...] = jnp.broadcast_to(b_ref[...], (_I4, 4 * _NUM_CLASS))

    acc = lax.dot_general(
        x_ref[0], w_ref[0], (((1,), (0,)), ((), ())),
        preferred_element_type=jnp.float32,
    )
    for j in range(1, _LB):
        acc += lax.dot_general(
            x_ref[j], w_ref[j], (((1,), (0,)), ((), ())),
            preferred_element_type=jnp.float32,
        )
    o_ref[...] += acc


def _tc_matmul(x3, w4, b4):
    return pl.pallas_call(
        _mm_body,
        grid=(_MAX_LEN // _LB,),
        in_specs=[
            pl.BlockSpec((_LB, _I4, 128), lambda i: (i, 0, 0)),
            pl.BlockSpec((_LB, 128, 4 * _NUM_CLASS), lambda i: (i, 0, 0)),
            pl.BlockSpec((1, 4 * _NUM_CLASS), lambda i: (0, 0)),
        ],
        out_specs=pl.BlockSpec((_I4, 4 * _NUM_CLASS), lambda i: (0, 0)),
        out_shape=jax.ShapeDtypeStruct((_I4, 4 * _NUM_CLASS), jnp.float32),
    )(x3, w4, b4)


@jax.jit
def kernel(text, emb_table, fc_w, fc_b):
    flat_idx = text.T.reshape(_TOTAL).astype(jnp.int32)
    tail = emb_table[_TCOLS * 128:, :].reshape(16, 128)
    tab_lin = _sc_transpose(emb_table.T, tail).reshape(_VOCAB, _EMBED)
    rows = _sc_gather(flat_idx, tab_lin)
    x3 = rows.reshape(_MAX_LEN, _I4, 128)
    wl = fc_w.reshape(_NUM_CLASS, _MAX_LEN, _EMBED).transpose(1, 2, 0)
    eye4 = jnp.eye(4, dtype=jnp.float32)
    w4 = (wl[:, None, :, None, :] * eye4[None, :, None, :, None]).reshape(
        _MAX_LEN, 128, 4 * _NUM_CLASS)
    b4 = jnp.tile(fc_b, 4).reshape(1, 4 * _NUM_CLASS)
    out4 = _tc_matmul(x3, w4, b4)
    return out4.reshape(_BATCH, _NUM_CLASS)
